# Initial kernel scaffold; baseline (speedup 1.0000x reference)
import jax
import jax.numpy as jnp
from jax.experimental import pallas as pl


def kernel(x, edge_index, Wl1, bl1, Wr1, Wl2, bl2, Wr2, Wl3, bl3, Wr3):
    raise NotImplementedError("write your pallas kernel here")


# speedup vs baseline: 2.9214x; 2.9214x over previous
import functools

import jax
import jax.numpy as jnp
from jax import lax
from jax.experimental import pallas as pl
from jax.experimental.pallas import tpu as pltpu
from jax.experimental.pallas import tpu_sc as plsc

N = 10000
E = 160000
NPAD = 10240
LANE = 64
NTILE = 16
EPT = E // NTILE
BATCH = 128
NB = 80
EPT_PAD = NB * BATCH
ROWS_PER_TILE = NPAD // NTILE
NFLUSH = ROWS_PER_TILE // 128


def _make_sc_agg(nc: int, with_cnt: bool):
    npass = nc // 2
    mesh = plsc.VectorSubcoreMesh(core_axis_name="c", subcore_axis_name="s")

    out_type = [jax.ShapeDtypeStruct((NPAD, nc, LANE), jnp.float32)]
    scratch = [
        pltpu.VMEM((NB, BATCH), jnp.int32),
        pltpu.VMEM((NB, BATCH), jnp.int32),
        pltpu.VMEM((NB, BATCH), jnp.int32),
        pltpu.VMEM((BATCH, LANE), jnp.float32),
        pltpu.VMEM((BATCH, LANE), jnp.float32),
        pltpu.VMEM_SHARED((NPAD, LANE), jnp.float32),
        pltpu.SemaphoreType.DMA,
    ]
    if with_cnt:
        out_type.append(jax.ShapeDtypeStruct((NPAD, LANE), jnp.float32))

    def body(hflat, srcs, dsts, *refs):
        if with_cnt:
            out, cnt_out, src_v, sidx_v, dst_v, rows_v, zero_v, agg_sh, \
                sem = refs
        else:
            out, src_v, sidx_v, dst_v, rows_v, zero_v, agg_sh, sem = refs
        core = lax.axis_index("c")
        sub = lax.axis_index("s")
        row0 = sub * ROWS_PER_TILE

        z16 = jnp.zeros((16,), jnp.float32)

        def memset_row(i, carry):
            for j in range(LANE // 16):
                zero_v[i, pl.ds(j * 16, 16)] = z16
            return carry

        lax.fori_loop(0, BATCH, memset_row, 0)

        pltpu.sync_copy(dsts.at[sub], dst_v)
        pltpu.sync_copy(srcs.at[sub], src_v)

        for p in range(npass):
            chunk = core * npass + p

            def scale_row(i, carry):
                for j in range(BATCH // 16):
                    s = src_v[i, pl.ds(j * 16, 16)]
                    sidx_v[i, pl.ds(j * 16, 16)] = s * nc + chunk
                return carry

            lax.fori_loop(0, NB, scale_row, 0)
            for j in range(NFLUSH):
                pltpu.sync_copy(zero_v,
                                agg_sh.at[pl.ds(row0 + j * 128, 128)])
            plsc.subcore_barrier()

            def edge_batch(b, carry):
                pltpu.async_copy(hflat.at[sidx_v.at[b]], rows_v, sem).wait()
                pltpu.sync_copy(rows_v, agg_sh.at[dst_v.at[b]], add=True)
                return carry

            lax.fori_loop(0, NB, edge_batch, 0)
            plsc.subcore_barrier()

            for j in range(NFLUSH):
                r = row0 + j * 128
                pltpu.sync_copy(agg_sh.at[pl.ds(r, 128)], rows_v)
                pltpu.sync_copy(rows_v, out.at[pl.ds(r, 128), chunk])

        if with_cnt:
            for j in range(NFLUSH):
                pltpu.sync_copy(zero_v,
                                agg_sh.at[pl.ds(row0 + j * 128, 128)])
            o16 = jnp.ones((16,), jnp.float32)

            def ones_row(i, carry):
                for j in range(LANE // 16):
                    rows_v[i, pl.ds(j * 16, 16)] = o16
                return carry

            lax.fori_loop(0, BATCH, ones_row, 0)
            plsc.subcore_barrier()

            def cnt_batch(b, carry):
                pltpu.sync_copy(rows_v, agg_sh.at[dst_v.at[b]], add=True)
                return carry

            lax.fori_loop(0, NB, cnt_batch, 0)
            plsc.subcore_barrier()

            @pl.when(core == 0)
            def _flush_cnt():
                for j in range(NFLUSH):
                    r = row0 + j * 128
                    pltpu.sync_copy(agg_sh.at[pl.ds(r, 128)], zero_v)
                    pltpu.sync_copy(zero_v, cnt_out.at[pl.ds(r, 128)])

    return pl.kernel(body, out_type=tuple(out_type) if with_cnt else out_type[0],
                     mesh=mesh, scratch_types=scratch,
                     compiler_params=pltpu.CompilerParams(
                         use_tc_tiling_on_sc=False))


_BLK = 512
_GRID = NPAD // _BLK


def _row_spec(k):
    return pl.BlockSpec((_BLK, k), lambda i: (i, 0))


def _full_spec(a, b):
    return pl.BlockSpec((a, b), lambda i: (0, 0))


def _tc_sage_body(agg_ref, h_ref, cnt_ref, wl_ref, wr_ref, b_ref, o_ref, *,
                  relu):
    inv = 1.0 / jnp.maximum(cnt_ref[:, 0:1], 1.0)
    mean = agg_ref[...] * inv
    acc = lax.dot_general(mean, wl_ref[...], (((1,), (1,)), ((), ())),
                          preferred_element_type=jnp.float32)
    acc += lax.dot_general(h_ref[...], wr_ref[...], (((1,), (1,)), ((), ())),
                           preferred_element_type=jnp.float32)
    acc += b_ref[...]
    if relu:
        acc = jnp.maximum(acc, 0.0)
    o_ref[...] = acc


def _tc_sage(agg, h, cnt128, Wl, Wr, b, relu):
    fo, k = Wl.shape
    return pl.pallas_call(
        functools.partial(_tc_sage_body, relu=relu),
        grid=(_GRID,),
        in_specs=[_row_spec(k), _row_spec(k), _row_spec(LANE),
                  _full_spec(fo, k), _full_spec(fo, k), _full_spec(1, fo)],
        out_specs=_row_spec(fo),
        out_shape=jax.ShapeDtypeStruct((NPAD, fo), jnp.float32),
    )(agg, h, cnt128, Wl, Wr, b.reshape(1, fo))


def _tc_sage_fused_body(agg_ref, h_ref, cnt_ref, wl_ref, wr_ref, b_ref,
                        wnext_ref, o_ref, y_ref):
    inv = 1.0 / jnp.maximum(cnt_ref[:, 0:1], 1.0)
    mean = agg_ref[...] * inv
    acc = lax.dot_general(mean, wl_ref[...], (((1,), (1,)), ((), ())),
                          preferred_element_type=jnp.float32)
    acc += lax.dot_general(h_ref[...], wr_ref[...], (((1,), (1,)), ((), ())),
                           preferred_element_type=jnp.float32)
    acc = jnp.maximum(acc + b_ref[...], 0.0)
    o_ref[...] = acc
    y_ref[...] = lax.dot_general(acc, wnext_ref[...], (((1,), (1,)), ((), ())),
                                 preferred_element_type=jnp.float32)


def _tc_sage_fused(agg, h, cnt128, Wl, Wr, b, Wnext):
    fo, k = Wl.shape
    fn = Wnext.shape[0]
    return pl.pallas_call(
        _tc_sage_fused_body,
        grid=(_GRID,),
        in_specs=[_row_spec(k), _row_spec(k), _row_spec(LANE),
                  _full_spec(fo, k), _full_spec(fo, k), _full_spec(1, fo),
                  _full_spec(fn, fo)],
        out_specs=[_row_spec(fo), _row_spec(fn)],
        out_shape=[jax.ShapeDtypeStruct((NPAD, fo), jnp.float32),
                   jax.ShapeDtypeStruct((NPAD, fn), jnp.float32)],
    )(agg, h, cnt128, Wl, Wr, b.reshape(1, fo), Wnext)


def _tc_final_body(aggy_ref, h_ref, cnt_ref, wr_ref, b_ref, o_ref):
    inv = 1.0 / jnp.maximum(cnt_ref[:, 0:1], 1.0)
    acc = aggy_ref[...] * inv
    acc += lax.dot_general(h_ref[...], wr_ref[...], (((1,), (1,)), ((), ())),
                           preferred_element_type=jnp.float32)
    o_ref[...] = acc + b_ref[...]


def _tc_final(aggy, h, cnt128, Wr, b):
    fo, k = Wr.shape
    return pl.pallas_call(
        _tc_final_body,
        grid=(_GRID,),
        in_specs=[_row_spec(fo), _row_spec(k), _row_spec(LANE),
                  _full_spec(fo, k), _full_spec(1, fo)],
        out_specs=_row_spec(fo),
        out_shape=jax.ShapeDtypeStruct((NPAD, fo), jnp.float32),
    )(aggy, h, cnt128, Wr, b.reshape(1, fo))


_NC1 = 256 // LANE
_NC2 = 512 // LANE
_sc_agg_narrow_cnt = _make_sc_agg(_NC1, with_cnt=True)
_sc_agg_wide = _make_sc_agg(_NC2, with_cnt=False)
_sc_agg_narrow = _make_sc_agg(_NC1, with_cnt=False)


def kernel(x, edge_index, Wl1, bl1, Wr1, Wl2, bl2, Wr2, Wl3, bl3, Wr3):
    src = edge_index[0].astype(jnp.int32)
    dst = edge_index[1].astype(jnp.int32)

    pad = EPT_PAD - EPT
    srcp = jnp.pad(src.reshape(NTILE, EPT), ((0, 0), (0, pad)))
    dstp = jnp.pad(dst.reshape(NTILE, EPT), ((0, 0), (0, pad)),
                   constant_values=N)
    dsts = dstp.reshape(NTILE, NB, BATCH)
    srcs = srcp.reshape(NTILE, NB, BATCH)

    xp = jnp.pad(x, ((0, NPAD - N), (0, 0)))

    agg1, cnt128 = _sc_agg_narrow_cnt(xp.reshape(NPAD * _NC1, LANE), srcs,
                                      dsts)
    h1 = _tc_sage(agg1.reshape(NPAD, 256), xp, cnt128, Wl1, Wr1, bl1,
                  relu=True)

    agg2 = _sc_agg_wide(h1.reshape(NPAD * _NC2, LANE), srcs, dsts)
    h2, y3 = _tc_sage_fused(agg2.reshape(NPAD, 512), h1, cnt128, Wl2, Wr2,
                            bl2, Wl3)

    agg3 = _sc_agg_narrow(y3.reshape(NPAD * _NC1, LANE), srcs, dsts)
    out = _tc_final(agg3.reshape(NPAD, 256), h2, cnt128, Wr3, bl3)
    return out[:N]
